# baseline (speedup 1.0000x reference)
import functools

import jax
import jax.numpy as jnp
from jax import lax
from jax.experimental import pallas as pl
from jax.experimental.pallas import tpu as pltpu
from jax.experimental.pallas import tpu_sc as plsc

NC, NS = 2, 16
N = 10000
D = 128
E = 320000
EPT = E // NS
CH = 32
NCHUNK = -(-EPT // CH) + (-(-EPT // CH)) % 2
EPP = NCHUNK * CH


_BM = 1000


def _proj_body(x_ref, w_ref, b_ref, wr_ref, blr_ref, h_ref, hw_ref):
    acc = jnp.dot(x_ref[0], w_ref[0], preferred_element_type=jnp.float32)
    h = jnp.maximum(acc + b_ref[0], 0.0)
    h_ref[0] = h
    hw_ref[0] = (jnp.dot(h, wr_ref[0], preferred_element_type=jnp.float32)
                 + blr_ref[0])


def _project(x, W, b, Wr, bl):
    grid = (2, N // _BM)
    return pl.pallas_call(
        _proj_body,
        grid=grid,
        in_specs=[
            pl.BlockSpec((1, _BM, D), lambda t, i: (t, i, 0)),
            pl.BlockSpec((1, D, D), lambda t, i: (t, 0, 0)),
            pl.BlockSpec((1, 1, D), lambda t, i: (t, 0, 0)),
            pl.BlockSpec((1, D, D), lambda t, i: (t, 0, 0)),
            pl.BlockSpec((1, 1, D), lambda t, i: (t, 0, 0)),
        ],
        out_specs=[
            pl.BlockSpec((1, _BM, D), lambda t, i: (t, i, 0)),
            pl.BlockSpec((1, _BM, D), lambda t, i: (t, i, 0)),
        ],
        out_shape=[
            jax.ShapeDtypeStruct((2, N, D), jnp.float32),
            jax.ShapeDtypeStruct((2, N, D), jnp.float32),
        ],
    )(x, W, b, Wr, bl)


ACC_ROWS = N + 16
ZROWS = ACC_ROWS // NS
CNT_ROWS = N + 16


def _seg_sum_body(edges_hbm, h_hbm, zero_hbm, acc_out, cnt_out,
                  eidx, rows2, cnt_v, acc_sh,
                  sem_g0, sem_g1, sem_s0, sem_s1, sem_i):
    c = lax.axis_index("c")
    s = lax.axis_index("s")
    tid = c * NS + s
    ngrp = NCHUNK // 2

    def _idx_fire(g):
        gc = jnp.minimum(g, ngrp - 1)
        pltpu.async_copy(edges_hbm.at[tid * ngrp + gc],
                         eidx.at[pl.ds((g % 4) * 4, 4)], sem_i)

    def _idx_wait():
        pltpu.make_async_copy(
            edges_hbm.at[tid * ngrp], eidx.at[pl.ds(0, 4)], sem_i).wait()

    def _zero_cnt(i, carry):
        cnt_v[pl.ds(i * 16, 16)] = jnp.zeros((16,), jnp.float32)
        return carry
    lax.fori_loop(0, CNT_ROWS // 16, _zero_cnt, 0)

    pltpu.sync_copy(zero_hbm, acc_sh.at[pl.ds(s * ZROWS, ZROWS)])
    plsc.subcore_barrier()

    def _cnts(r):
        for v in range(CH // 16):
            d0 = eidx[r, pl.ds(v * 16, 16)]
            plsc.addupdate_scatter(cnt_v, [d0], jnp.ones((16,), jnp.float32))

    _idx_fire(jnp.int32(0))
    _idx_wait()
    _idx_fire(jnp.int32(1))
    pltpu.async_copy(h_hbm.at[eidx.at[0]], rows2.at[0], sem_g0)

    def _group(g, carry):
        r0 = (g % 4) * 4
        rn = ((g + 1) % 4) * 4
        _idx_wait()
        _idx_fire(g + 2)
        @pl.when(g > 0)
        def _():
            pltpu.make_async_copy(
                rows2.at[1], acc_sh.at[eidx.at[3]], sem_s1).wait()
        pltpu.async_copy(h_hbm.at[eidx.at[r0 + 1]], rows2.at[1], sem_g1)
        _cnts(r0 + 2)
        pltpu.make_async_copy(
            h_hbm.at[eidx.at[r0]], rows2.at[0], sem_g0).wait()
        pltpu.async_copy(
            rows2.at[0], acc_sh.at[eidx.at[r0 + 2]], sem_s0, add=True)
        pltpu.make_async_copy(
            rows2.at[0], acc_sh.at[eidx.at[2]], sem_s0).wait()
        pltpu.async_copy(h_hbm.at[eidx.at[rn]], rows2.at[0], sem_g0)
        _cnts(r0 + 3)
        pltpu.make_async_copy(
            h_hbm.at[eidx.at[r0 + 1]], rows2.at[1], sem_g1).wait()
        pltpu.async_copy(
            rows2.at[1], acc_sh.at[eidx.at[r0 + 3]], sem_s1, add=True)
        return carry

    lax.fori_loop(0, ngrp, _group, 0)
    _idx_wait()
    pltpu.make_async_copy(
        h_hbm.at[eidx.at[0]], rows2.at[0], sem_g0).wait()
    pltpu.make_async_copy(
        rows2.at[1], acc_sh.at[eidx.at[3]], sem_s1).wait()
    plsc.subcore_barrier()
    pltpu.sync_copy(acc_sh.at[pl.ds(s * ZROWS, ZROWS)], acc_out.at[c, s])
    pltpu.sync_copy(cnt_v, cnt_out.at[c, s])


@functools.cache
def _seg_sum_kernel():
    mesh = plsc.VectorSubcoreMesh(
        core_axis_name="c", subcore_axis_name="s",
        num_cores=NC, num_subcores=NS)
    return pl.kernel(
        _seg_sum_body,
        out_type=[
            jax.ShapeDtypeStruct((NC, NS, ZROWS, D), jnp.float32),
            jax.ShapeDtypeStruct((NC, NS, CNT_ROWS), jnp.float32),
        ],
        name="seg_sum_sc",
        mesh=mesh,
        compiler_params=pltpu.CompilerParams(needs_layout_passes=False),
        scratch_types=[
            pltpu.VMEM((16, CH), jnp.int32),
            pltpu.VMEM((2, CH, D), jnp.float32),
            pltpu.VMEM((CNT_ROWS,), jnp.float32),
            pltpu.VMEM_SHARED((ACC_ROWS, D), jnp.float32),
            pltpu.SemaphoreType.DMA,
            pltpu.SemaphoreType.DMA,
            pltpu.SemaphoreType.DMA,
            pltpu.SemaphoreType.DMA,
            pltpu.SemaphoreType.DMA,
        ],
    )


def _seg_sum(edges, h2, zeros):
    acc, cnt = _seg_sum_kernel()(edges, h2, zeros)
    return acc.reshape(NC, NS * ZROWS, D), cnt


def _out_body(sp_ref, cnt_ref, hw_ref, wl_ref, o_ref):
    i = pl.program_id(1)
    cnt = jnp.sum(cnt_ref[0], axis=1)
    swl = jnp.dot(sp_ref[0], wl_ref[0], preferred_element_type=jnp.float32)
    o = swl / jnp.maximum(cnt, 1.0)[:, None] + hw_ref[0]
    part = jnp.sum(jnp.maximum(o, 0.0), axis=0)

    @pl.when(i == 0)
    def _init():
        o_ref[0] = jnp.zeros_like(o_ref[0])

    o_ref[0, 0] += part


def _finalize(Sp, cntp, hw, Wl):
    grid = (2, N // _BM)
    return pl.pallas_call(
        _out_body,
        grid=grid,
        in_specs=[
            pl.BlockSpec((1, _BM, D), lambda t, i: (t, i, 0)),
            pl.BlockSpec((1, _BM, NS), lambda t, i: (t, i, 0)),
            pl.BlockSpec((1, _BM, D), lambda t, i: (t, i, 0)),
            pl.BlockSpec((1, D, D), lambda t, i: (t, 0, 0)),
        ],
        out_specs=pl.BlockSpec((1, 8, D), lambda t, i: (t, 0, 0)),
        out_shape=jax.ShapeDtypeStruct((2, 8, D), jnp.float32),
    )(Sp, cntp, hw, Wl)[:, 0, :]


def kernel(x_user, x_item, edge_index_ui, edge_index_iu,
           W_user, b_user, W_item, b_item,
           Wl_ui, bl_ui, Wr_ui, Wl_iu, bl_iu, Wr_iu):
    x = jnp.stack([x_user, x_item])
    W = jnp.stack([W_user, W_item])
    b = jnp.stack([b_user, b_item])[:, None, :]
    Wr = jnp.stack([Wr_iu, Wr_ui])
    bl = jnp.stack([bl_iu, bl_ui])[:, None, :]
    h, hw = _project(x, W, b, Wr, bl)
    h2 = h.reshape(2 * N, D)

    ngrp = NCHUNK // 2

    def _tile_pad(a, val):
        a = a.reshape(NC * NS, EPT)
        return jnp.pad(a, ((0, 0), (0, EPP - EPT)), constant_values=val)

    src = _tile_pad(jnp.concatenate([
        edge_index_iu[0].astype(jnp.int32) + N,
        edge_index_ui[0].astype(jnp.int32),
    ]), 0).reshape(NC * NS, ngrp, 2, CH)
    dst = _tile_pad(jnp.concatenate([
        edge_index_iu[1].astype(jnp.int32),
        edge_index_ui[1].astype(jnp.int32),
    ]), N).reshape(NC * NS, ngrp, 2, CH)
    edges = jnp.concatenate([src, dst], axis=2).reshape(
        NC * NS * ngrp, 4, CH)
    zeros = jnp.zeros((ZROWS, D), jnp.float32)

    acc, cnt = _seg_sum(edges, h2, zeros)

    Wl = jnp.stack([Wl_iu, Wl_ui])
    cnt_t = jnp.swapaxes(cnt, 1, 2)
    pooled = _finalize(acc, cnt_t, hw, Wl)
    return (pooled / N).reshape(2 * D)


# speedup vs baseline: 1.0952x; 1.0952x over previous
import functools

import jax
import jax.numpy as jnp
from jax import lax
from jax.experimental import pallas as pl
from jax.experimental.pallas import tpu as pltpu
from jax.experimental.pallas import tpu_sc as plsc

NC, NS = 2, 16
N = 10000
D = 128
E = 320000
EPT = E // NS
CH = 48
NCHUNK = -(-EPT // CH) + (-(-EPT // CH)) % 2
EPP = NCHUNK * CH


_BM = 1000


def _proj_body(x_ref, w_ref, b_ref, wr_ref, blr_ref, h_ref, hw_ref):
    acc = jnp.dot(x_ref[0], w_ref[0], preferred_element_type=jnp.float32)
    h = jnp.maximum(acc + b_ref[0], 0.0)
    h_ref[0] = h
    hw_ref[0] = (jnp.dot(h, wr_ref[0], preferred_element_type=jnp.float32)
                 + blr_ref[0])


def _project(x, W, b, Wr, bl):
    grid = (2, N // _BM)
    return pl.pallas_call(
        _proj_body,
        grid=grid,
        in_specs=[
            pl.BlockSpec((1, _BM, D), lambda t, i: (t, i, 0)),
            pl.BlockSpec((1, D, D), lambda t, i: (t, 0, 0)),
            pl.BlockSpec((1, 1, D), lambda t, i: (t, 0, 0)),
            pl.BlockSpec((1, D, D), lambda t, i: (t, 0, 0)),
            pl.BlockSpec((1, 1, D), lambda t, i: (t, 0, 0)),
        ],
        out_specs=[
            pl.BlockSpec((1, _BM, D), lambda t, i: (t, i, 0)),
            pl.BlockSpec((1, _BM, D), lambda t, i: (t, i, 0)),
        ],
        out_shape=[
            jax.ShapeDtypeStruct((2, N, D), jnp.float32),
            jax.ShapeDtypeStruct((2, N, D), jnp.float32),
        ],
    )(x, W, b, Wr, bl)


ACC_ROWS = N + 16
ZROWS = ACC_ROWS // NS
CNT_ROWS = N + 16


def _seg_sum_body(src_hbm, dst_hbm, h_hbm, zero_hbm, acc_out, cnt_out,
                  sidx, didx, rows2, cnt_v, acc_sh,
                  sem_g0, sem_g1, sem_s0, sem_s1, sem_i):
    c = lax.axis_index("c")
    s = lax.axis_index("s")
    tid = c * NS + s
    ngrp = NCHUNK // 2

    def _idx_fire(g):
        gc = jnp.minimum(g, ngrp - 1)
        r = (g % 4) * 2
        pltpu.async_copy(src_hbm.at[tid * ngrp + gc],
                         sidx.at[pl.ds(r, 2)], sem_i)
        pltpu.async_copy(dst_hbm.at[tid * ngrp + gc],
                         didx.at[pl.ds(r, 2)], sem_i)

    def _idx_wait():
        pltpu.make_async_copy(
            src_hbm.at[tid * ngrp], sidx.at[pl.ds(0, 2)], sem_i).wait()
        pltpu.make_async_copy(
            dst_hbm.at[tid * ngrp], didx.at[pl.ds(0, 2)], sem_i).wait()

    def _zero_cnt(i, carry):
        cnt_v[pl.ds(i * 16, 16)] = jnp.zeros((16,), jnp.float32)
        return carry
    lax.fori_loop(0, CNT_ROWS // 16, _zero_cnt, 0)

    pltpu.sync_copy(zero_hbm, acc_sh.at[pl.ds(s * ZROWS, ZROWS)])
    plsc.subcore_barrier()

    def _cnts(r):
        for v in range(CH // 16):
            d0 = didx[r, pl.ds(v * 16, 16)]
            plsc.addupdate_scatter(cnt_v, [d0], jnp.ones((16,), jnp.float32))

    _idx_fire(jnp.int32(0))
    _idx_wait()
    _idx_fire(jnp.int32(1))
    pltpu.async_copy(h_hbm.at[sidx.at[0]], rows2.at[0], sem_g0)

    def _group(g, carry):
        r0 = (g % 4) * 2
        rn = ((g + 1) % 4) * 2
        _idx_wait()
        _idx_fire(g + 2)
        @pl.when(g > 0)
        def _():
            pltpu.make_async_copy(
                rows2.at[1], acc_sh.at[didx.at[1]], sem_s1).wait()
        pltpu.async_copy(h_hbm.at[sidx.at[r0 + 1]], rows2.at[1], sem_g1)
        _cnts(r0)
        pltpu.make_async_copy(
            h_hbm.at[sidx.at[r0]], rows2.at[0], sem_g0).wait()
        pltpu.async_copy(
            rows2.at[0], acc_sh.at[didx.at[r0]], sem_s0, add=True)
        pltpu.make_async_copy(
            rows2.at[0], acc_sh.at[didx.at[0]], sem_s0).wait()
        pltpu.async_copy(h_hbm.at[sidx.at[rn]], rows2.at[0], sem_g0)
        _cnts(r0 + 1)
        pltpu.make_async_copy(
            h_hbm.at[sidx.at[r0 + 1]], rows2.at[1], sem_g1).wait()
        pltpu.async_copy(
            rows2.at[1], acc_sh.at[didx.at[r0 + 1]], sem_s1, add=True)
        return carry

    lax.fori_loop(0, ngrp, _group, 0)
    _idx_wait()
    pltpu.make_async_copy(
        h_hbm.at[sidx.at[0]], rows2.at[0], sem_g0).wait()
    pltpu.make_async_copy(
        rows2.at[1], acc_sh.at[didx.at[1]], sem_s1).wait()
    plsc.subcore_barrier()
    pltpu.sync_copy(acc_sh.at[pl.ds(s * ZROWS, ZROWS)], acc_out.at[c, s])
    pltpu.sync_copy(cnt_v, cnt_out.at[c, s])


@functools.cache
def _seg_sum_kernel():
    mesh = plsc.VectorSubcoreMesh(
        core_axis_name="c", subcore_axis_name="s",
        num_cores=NC, num_subcores=NS)
    return pl.kernel(
        _seg_sum_body,
        out_type=[
            jax.ShapeDtypeStruct((NC, NS, ZROWS, D), jnp.float32),
            jax.ShapeDtypeStruct((NC, NS, CNT_ROWS), jnp.float32),
        ],
        name="seg_sum_sc",
        mesh=mesh,
        compiler_params=pltpu.CompilerParams(needs_layout_passes=False),
        scratch_types=[
            pltpu.VMEM((8, CH), jnp.int32),
            pltpu.VMEM((8, CH), jnp.int32),
            pltpu.VMEM((2, CH, D), jnp.float32),
            pltpu.VMEM((CNT_ROWS,), jnp.float32),
            pltpu.VMEM_SHARED((ACC_ROWS, D), jnp.float32),
            pltpu.SemaphoreType.DMA,
            pltpu.SemaphoreType.DMA,
            pltpu.SemaphoreType.DMA,
            pltpu.SemaphoreType.DMA,
            pltpu.SemaphoreType.DMA,
        ],
    )


def _seg_sum(src, dst, h2, zeros):
    acc, cnt = _seg_sum_kernel()(src, dst, h2, zeros)
    return acc.reshape(NC, NS * ZROWS, D), cnt


def _out_body(sp_ref, cnt_ref, hw_ref, wl_ref, o_ref):
    i = pl.program_id(1)
    cnt = jnp.sum(cnt_ref[0], axis=1)
    swl = jnp.dot(sp_ref[0], wl_ref[0], preferred_element_type=jnp.float32)
    o = swl / jnp.maximum(cnt, 1.0)[:, None] + hw_ref[0]
    part = jnp.sum(jnp.maximum(o, 0.0), axis=0)

    @pl.when(i == 0)
    def _init():
        o_ref[0] = jnp.zeros_like(o_ref[0])

    o_ref[0, 0] += part


def _finalize(Sp, cntp, hw, Wl):
    grid = (2, N // _BM)
    return pl.pallas_call(
        _out_body,
        grid=grid,
        in_specs=[
            pl.BlockSpec((1, _BM, D), lambda t, i: (t, i, 0)),
            pl.BlockSpec((1, _BM, NS), lambda t, i: (t, i, 0)),
            pl.BlockSpec((1, _BM, D), lambda t, i: (t, i, 0)),
            pl.BlockSpec((1, D, D), lambda t, i: (t, 0, 0)),
        ],
        out_specs=pl.BlockSpec((1, 8, D), lambda t, i: (t, 0, 0)),
        out_shape=jax.ShapeDtypeStruct((2, 8, D), jnp.float32),
    )(Sp, cntp, hw, Wl)[:, 0, :]


def kernel(x_user, x_item, edge_index_ui, edge_index_iu,
           W_user, b_user, W_item, b_item,
           Wl_ui, bl_ui, Wr_ui, Wl_iu, bl_iu, Wr_iu):
    x = jnp.stack([x_user, x_item])
    W = jnp.stack([W_user, W_item])
    b = jnp.stack([b_user, b_item])[:, None, :]
    Wr = jnp.stack([Wr_iu, Wr_ui])
    bl = jnp.stack([bl_iu, bl_ui])[:, None, :]
    h, hw = _project(x, W, b, Wr, bl)
    h2 = h.reshape(2 * N, D)

    ngrp = NCHUNK // 2
    tail = NS * EPP - E
    src = jnp.concatenate([
        edge_index_iu[0].astype(jnp.int32) + N,
        jnp.zeros((tail,), jnp.int32),
        edge_index_ui[0].astype(jnp.int32),
        jnp.zeros((tail,), jnp.int32),
    ]).reshape(NC * NS * ngrp, 2, CH)
    dst = jnp.concatenate([
        edge_index_iu[1].astype(jnp.int32),
        jnp.full((tail,), N, jnp.int32),
        edge_index_ui[1].astype(jnp.int32),
        jnp.full((tail,), N, jnp.int32),
    ]).reshape(NC * NS * ngrp, 2, CH)
    zeros = jnp.zeros((ZROWS, D), jnp.float32)

    acc, cnt = _seg_sum(src, dst, h2, zeros)

    Wl = jnp.stack([Wl_iu, Wl_ui])
    cnt_t = jnp.swapaxes(cnt, 1, 2)
    pooled = _finalize(acc, cnt_t, hw, Wl)
    return (pooled / N).reshape(2 * D)
